# Initial kernel scaffold; baseline (speedup 1.0000x reference)
import jax
import jax.numpy as jnp
from jax.experimental import pallas as pl


def kernel(x, edge_index, edge_attr, params):
    raise NotImplementedError("write your pallas kernel here")


# speedup vs baseline: 3.5192x; 3.5192x over previous
import functools

import jax
import jax.numpy as jnp
from jax import lax
from jax.experimental import pallas as pl
from jax.experimental.pallas import tpu as pltpu
from jax.experimental.pallas import tpu_sc as plsc

HID = 64
ACC_D = 128
NPAD = 10240


def _row_mlp_call(X, W1, b1, W2, b2, bs, want_max=False):
    R, din = X.shape
    dh = W1.shape[1]
    dout = W2.shape[1]
    grid = R // bs

    def body(x_ref, w1_ref, b1_ref, w2_ref, b2_ref, o_ref, *maybe_mx):
        h = jnp.maximum(
            jnp.dot(x_ref[...], w1_ref[...], preferred_element_type=jnp.float32)
            + b1_ref[...], 0.0)
        o = jnp.dot(h, w2_ref[...], preferred_element_type=jnp.float32) + b2_ref[...]
        o_ref[...] = o
        if maybe_mx:
            mx_ref = maybe_mx[0]

            @pl.when(pl.program_id(0) == 0)
            def _():
                mx_ref[...] = jnp.full((8, 128), -jnp.inf, jnp.float32)

            mx_ref[...] = jnp.maximum(mx_ref[...], jnp.max(o))

    out_shape = [jax.ShapeDtypeStruct((R, dout), jnp.float32)]
    out_specs = [pl.BlockSpec((bs, dout), lambda i: (i, 0))]
    if want_max:
        out_shape.append(jax.ShapeDtypeStruct((8, 128), jnp.float32))
        out_specs.append(pl.BlockSpec((8, 128), lambda i: (0, 0)))
    return pl.pallas_call(
        body,
        grid=(grid,),
        in_specs=[
            pl.BlockSpec((bs, din), lambda i: (i, 0)),
            pl.BlockSpec((din, dh), lambda i: (0, 0)),
            pl.BlockSpec((1, dh), lambda i: (0, 0)),
            pl.BlockSpec((dh, dout), lambda i: (0, 0)),
            pl.BlockSpec((1, dout), lambda i: (0, 0)),
        ],
        out_specs=out_specs if want_max else out_specs[0],
        out_shape=out_shape if want_max else out_shape[0],
    )(X, W1, b1.reshape(1, dh), W2, b2.reshape(1, dout))


def _ln_relu_call(X, g, b, bs, pad_to=None):
    R, d = X.shape
    dout = pad_to or d
    grid = R // bs

    def body(x_ref, g_ref, b_ref, o_ref, mx_ref):
        x = x_ref[...]
        mu = jnp.mean(x, axis=-1, keepdims=True)
        var = jnp.mean((x - mu) ** 2, axis=-1, keepdims=True)
        o = jnp.maximum((x - mu) / jnp.sqrt(var + 1e-5) * g_ref[...] + b_ref[...], 0.0)
        if dout > d:
            o_ref[...] = jnp.concatenate(
                [o, jnp.zeros((o.shape[0], dout - d), jnp.float32)], axis=1)
        else:
            o_ref[...] = o

        @pl.when(pl.program_id(0) == 0)
        def _():
            mx_ref[...] = jnp.full((8, 128), -jnp.inf, jnp.float32)

        mx_ref[...] = jnp.maximum(mx_ref[...], jnp.max(o))

    return pl.pallas_call(
        body,
        grid=(grid,),
        in_specs=[
            pl.BlockSpec((bs, d), lambda i: (i, 0)),
            pl.BlockSpec((1, d), lambda i: (0, 0)),
            pl.BlockSpec((1, d), lambda i: (0, 0)),
        ],
        out_specs=[
            pl.BlockSpec((bs, dout), lambda i: (i, 0)),
            pl.BlockSpec((8, 128), lambda i: (0, 0)),
        ],
        out_shape=[
            jax.ShapeDtypeStruct((R, dout), jnp.float32),
            jax.ShapeDtypeStruct((8, 128), jnp.float32),
        ],
    )(X, g.reshape(1, d), b.reshape(1, d))


def _msg_call(G, Eh, scal, bs, with_g):
    R, d = Eh.shape
    grid = R // bs

    def body(*refs):
        if with_g:
            g_ref, e_ref, s_ref, o_ref = refs
            m = jnp.maximum(g_ref[...][:, :d] + e_ref[...], 0.0) + 1e-7
        else:
            e_ref, s_ref, o_ref = refs
            m = jnp.maximum(e_ref[...], 0.0) + 1e-7
        ex = jnp.exp(s_ref[0] * m - s_ref[1])
        o_ref[...] = jnp.concatenate([ex, m * ex], axis=1)

    in_specs = [pl.BlockSpec((bs, d), lambda i: (i, 0))]
    args = [Eh, scal]
    if with_g:
        gw = G.shape[1]
        in_specs = [pl.BlockSpec((bs, gw), lambda i: (i, 0))] + in_specs
        args = [G] + args
    in_specs.append(pl.BlockSpec(memory_space=pltpu.SMEM))
    return pl.pallas_call(
        body,
        grid=(grid,),
        in_specs=in_specs,
        out_specs=pl.BlockSpec((bs, 2 * d), lambda i: (i, 0)),
        out_shape=jax.ShapeDtypeStruct((R, 2 * d), jnp.float32),
    )(*args)


def _agg_div_call(P2, bs):
    _, NP, D = P2.shape
    d = D // 2
    grid = NP // bs

    def body(p_ref, o_ref):
        a = p_ref[0] + p_ref[1]
        o_ref[...] = jnp.concatenate(
            [a[:, d:] / (a[:, :d] + 1e-16), jnp.zeros((a.shape[0], d), jnp.float32)],
            axis=1)

    return pl.pallas_call(
        body,
        grid=(grid,),
        in_specs=[pl.BlockSpec((2, bs, D), lambda i: (0, i, 0))],
        out_specs=pl.BlockSpec((bs, D), lambda i: (i, 0)),
        out_shape=jax.ShapeDtypeStruct((NP, D), jnp.float32),
    )(P2)


def _fin_call(A, B, base, W1, b1, W2, b2, bs):
    R = B.shape[0]
    d = W1.shape[0]
    aw, bw = A.shape[1], B.shape[1]
    dh = W1.shape[1]
    dout = W2.shape[1]
    grid = R // bs

    def body(*refs):
        if base is not None:
            a_ref, b_ref, r_ref, w1_ref, b1_ref, w2_ref, b2_ref, o_ref = refs
        else:
            a_ref, b_ref, w1_ref, b1_ref, w2_ref, b2_ref, o_ref = refs
        h = a_ref[...][:, :d] + b_ref[...][:, :d]
        hh = jnp.maximum(
            jnp.dot(h, w1_ref[...], preferred_element_type=jnp.float32)
            + b1_ref[...], 0.0)
        o = jnp.dot(hh, w2_ref[...], preferred_element_type=jnp.float32) + b2_ref[...]
        if base is not None:
            o = r_ref[...] + o
        o_ref[...] = o

    in_specs = [
        pl.BlockSpec((bs, aw), lambda i: (i, 0)),
        pl.BlockSpec((bs, bw), lambda i: (i, 0)),
    ]
    args = [A, B]
    if base is not None:
        in_specs.append(pl.BlockSpec((bs, dout), lambda i: (i, 0)))
        args.append(base)
    in_specs += [
        pl.BlockSpec((d, dh), lambda i: (0, 0)),
        pl.BlockSpec((1, dh), lambda i: (0, 0)),
        pl.BlockSpec((dh, dout), lambda i: (0, 0)),
        pl.BlockSpec((1, dout), lambda i: (0, 0)),
    ]
    args += [W1, b1.reshape(1, dh), W2, b2.reshape(1, dout)]
    return pl.pallas_call(
        body,
        grid=(grid,),
        in_specs=in_specs,
        out_specs=pl.BlockSpec((bs, dout), lambda i: (i, 0)),
        out_shape=jax.ShapeDtypeStruct((R, dout), jnp.float32),
    )(*args)


_CH = 128


def _gather_rows(table, idx):
    T, D = table.shape
    (E,) = idx.shape
    info = plsc.get_sparse_core_info()
    NC, NS = info.num_cores, info.num_subcores
    NW = NC * NS
    nch = E // _CH
    per_w = -(-nch // NW)
    mesh = plsc.VectorSubcoreMesh(core_axis_name="c", subcore_axis_name="s")

    @functools.partial(
        pl.kernel,
        mesh=mesh,
        out_type=jax.ShapeDtypeStruct((E, D), jnp.float32),
        scratch_types=[
            pltpu.VMEM((_CH,), jnp.int32),
            pltpu.VMEM((_CH, D), jnp.float32),
            pltpu.SemaphoreType.DMA,
        ],
    )
    def k(table_hbm, idx_hbm, out_hbm, idx_v, rows_v, sem):
        wid = lax.axis_index("s") * NC + lax.axis_index("c")
        for t in range(per_w):
            c = wid + NW * t

            @pl.when(c < nch)
            def _():
                base = c * _CH
                pltpu.sync_copy(idx_hbm.at[pl.ds(base, _CH)], idx_v)
                pltpu.async_copy(table_hbm.at[idx_v], rows_v, sem).wait()
                pltpu.sync_copy(rows_v, out_hbm.at[pl.ds(base, _CH)])

    return k(table, idx)


def _scatter_add_rows(idx, upd, zeros):
    E, D = upd.shape
    NP = zeros.shape[0]
    info = plsc.get_sparse_core_info()
    NC, NS = info.num_cores, info.num_subcores
    NW = NC * NS
    nch = E // _CH
    per_w = -(-nch // NW)
    stripe = NP // NS
    mesh = plsc.VectorSubcoreMesh(core_axis_name="c", subcore_axis_name="s")

    @functools.partial(
        pl.kernel,
        mesh=mesh,
        out_type=jax.ShapeDtypeStruct((NC, NP, D), jnp.float32),
        scratch_types=[
            pltpu.VMEM((_CH,), jnp.int32),
            pltpu.VMEM((_CH, D), jnp.float32),
            pltpu.VMEM_SHARED((NP, D), jnp.float32),
        ],
    )
    def k(zeros_hbm, idx_hbm, upd_hbm, out_hbm, idx_v, upd_v, acc_sh):
        cid = lax.axis_index("c")
        sid = lax.axis_index("s")
        wid = sid * NC + cid
        row0 = sid * stripe
        pltpu.sync_copy(zeros_hbm.at[pl.ds(row0, stripe)],
                        acc_sh.at[pl.ds(row0, stripe)])
        plsc.subcore_barrier()
        for t in range(per_w):
            c = wid + NW * t

            @pl.when(c < nch)
            def _():
                base = c * _CH
                pltpu.sync_copy(idx_hbm.at[pl.ds(base, _CH)], idx_v)
                pltpu.sync_copy(upd_hbm.at[pl.ds(base, _CH)], upd_v)
                pltpu.sync_copy(upd_v, acc_sh.at[idx_v], add=True)

        plsc.subcore_barrier()
        pltpu.sync_copy(acc_sh.at[pl.ds(row0, stripe)],
                        out_hbm.at[cid, pl.ds(row0, stripe)])

    return k(zeros, idx, upd)


def _pad_cols(W, to):
    return jnp.zeros((W.shape[0], to), jnp.float32).at[:, :W.shape[1]].set(W)


def _pad_vec(b, to):
    return jnp.zeros((to,), jnp.float32).at[:b.shape[0]].set(b)


def _stab_const(beta, bound):
    return jnp.maximum(beta * (jnp.maximum(bound, 0.0) + 1e-7), 0.0)


def kernel(x, edge_index, edge_attr, params):
    N = x.shape[0]
    E = edge_index.shape[1]
    src = edge_index[0].astype(jnp.int32)
    dst = edge_index[1].astype(jnp.int32)

    k = src * N + dst
    ks_ = jnp.sort(k)
    keep = jnp.concatenate([jnp.ones((1,), bool), ks_[1:] != ks_[:-1]])
    rank = jnp.cumsum(keep) - 1
    idx = jnp.where(keep, rank.astype(jnp.int32), E)
    uk = jnp.zeros((E,), jnp.int32).at[idx].set(ks_, mode="drop")
    valid = jnp.arange(E) < keep.sum()
    row_c = uk // N
    col_c = uk % N
    seg = jnp.where(valid, col_c, N).astype(jnp.int32)
    rowg = jnp.where(valid, row_c, N + 1).astype(jnp.int32)

    zeros_acc = jnp.zeros((NPAD, ACC_D), jnp.float32)
    bs_n, bs_e = 2000, 2000

    pn, pe = params["node_enc"], params["edge_enc"]
    W2np = _pad_cols(pn["W2"], 128)
    b2np = _pad_vec(pn["b2"], 128)
    v, vmax8 = _row_mlp_call(x, pn["W1"], pn["b1"], W2np, b2np,
                             bs_n, want_max=True)
    e, emax8 = _row_mlp_call(edge_attr, pe["W1"], pe["b1"], pe["W2"], pe["b2"],
                             bs_e, want_max=True)

    for i, lp in enumerate(params["layers"]):
        if i == 0:
            vh, eh = v, e
            vmax, emax = jnp.max(vmax8), jnp.max(emax8)
        else:
            vh, vm8 = _ln_relu_call(v, lp["v_ln_g"], lp["v_ln_b"], bs_n,
                                    pad_to=128)
            eh, em8 = _ln_relu_call(e, lp["e_ln_g"], lp["e_ln_b"], bs_e)
            vmax, emax = jnp.max(vm8), jnp.max(em8)

        g = _gather_rows(vh, src)
        cv = _stab_const(lp["v_beta"], vmax + emax)
        Pv = _msg_call(g, eh, jnp.stack([lp["v_beta"], cv]), bs_e, with_g=True)
        partv = _scatter_add_rows(dst, Pv, zeros_acc)
        aggv = _agg_div_call(partv, 640)
        vmlp = lp["v_mlp"]
        v_new = _fin_call(aggv, vh, v if i > 0 else None,
                          vmlp["W1"], vmlp["b1"], vmlp["W2"], vmlp["b2"], bs_n)

        ce = _stab_const(lp["e_beta"], emax)
        Pe = _msg_call(None, eh, jnp.stack([lp["e_beta"], ce]), bs_e, with_g=False)
        parte = _scatter_add_rows(seg, Pe, zeros_acc)
        agge = _agg_div_call(parte, 640)
        ge = _gather_rows(agge, rowg)
        emlp = lp["e_mlp"]
        e_new = _fin_call(ge, eh, e if i > 0 else None,
                          emlp["W1"], emlp["b1"], emlp["W2"], emlp["b2"], bs_e)

        v, e = v_new, e_new

    pd_, pq = params["node_dec"], params["edge_dec"]
    dout = pd_["W2"].shape[1]
    node_out = _row_mlp_call(v, pd_["W1"], pd_["b1"],
                             _pad_cols(pd_["W2"], 128), _pad_vec(pd_["b2"], 128),
                             bs_n)[:, :dout]
    edge_out = _row_mlp_call(e, pq["W1"], pq["b1"],
                             _pad_cols(pq["W2"], 128), _pad_vec(pq["b2"], 128),
                             bs_e)[:, :dout]
    return node_out, edge_out


# speedup vs baseline: 4.2809x; 1.2164x over previous
import functools

import jax
import jax.numpy as jnp
from jax import lax
from jax.experimental import pallas as pl
from jax.experimental.pallas import tpu as pltpu
from jax.experimental.pallas import tpu_sc as plsc

HID = 64
ACC_D = 128
NPAD = 10240


def _row_mlp_call(X, W1, b1, W2, b2, bs, want_max=False):
    R, din = X.shape
    dh = W1.shape[1]
    dout = W2.shape[1]
    grid = R // bs

    def body(x_ref, w1_ref, b1_ref, w2_ref, b2_ref, o_ref, *maybe_mx):
        h = jnp.maximum(
            jnp.dot(x_ref[...], w1_ref[...], preferred_element_type=jnp.float32)
            + b1_ref[...], 0.0)
        o = jnp.dot(h, w2_ref[...], preferred_element_type=jnp.float32) + b2_ref[...]
        o_ref[...] = o
        if maybe_mx:
            mx_ref = maybe_mx[0]

            @pl.when(pl.program_id(0) == 0)
            def _():
                mx_ref[...] = jnp.full((8, 128), -jnp.inf, jnp.float32)

            mx_ref[...] = jnp.maximum(mx_ref[...], jnp.max(o))

    out_shape = [jax.ShapeDtypeStruct((R, dout), jnp.float32)]
    out_specs = [pl.BlockSpec((bs, dout), lambda i: (i, 0))]
    if want_max:
        out_shape.append(jax.ShapeDtypeStruct((8, 128), jnp.float32))
        out_specs.append(pl.BlockSpec((8, 128), lambda i: (0, 0)))
    return pl.pallas_call(
        body,
        grid=(grid,),
        in_specs=[
            pl.BlockSpec((bs, din), lambda i: (i, 0)),
            pl.BlockSpec((din, dh), lambda i: (0, 0)),
            pl.BlockSpec((1, dh), lambda i: (0, 0)),
            pl.BlockSpec((dh, dout), lambda i: (0, 0)),
            pl.BlockSpec((1, dout), lambda i: (0, 0)),
        ],
        out_specs=out_specs if want_max else out_specs[0],
        out_shape=out_shape if want_max else out_shape[0],
    )(X, W1, b1.reshape(1, dh), W2, b2.reshape(1, dout))


def _ln_relu_call(X, g, b, bs, pad_to=None):
    R, d = X.shape
    dout = pad_to or d
    grid = R // bs

    def body(x_ref, g_ref, b_ref, o_ref, mx_ref):
        x = x_ref[...]
        mu = jnp.mean(x, axis=-1, keepdims=True)
        var = jnp.mean((x - mu) ** 2, axis=-1, keepdims=True)
        o = jnp.maximum((x - mu) / jnp.sqrt(var + 1e-5) * g_ref[...] + b_ref[...], 0.0)
        if dout > d:
            o_ref[...] = jnp.concatenate(
                [o, jnp.zeros((o.shape[0], dout - d), jnp.float32)], axis=1)
        else:
            o_ref[...] = o

        @pl.when(pl.program_id(0) == 0)
        def _():
            mx_ref[...] = jnp.full((8, 128), -jnp.inf, jnp.float32)

        mx_ref[...] = jnp.maximum(mx_ref[...], jnp.max(o))

    return pl.pallas_call(
        body,
        grid=(grid,),
        in_specs=[
            pl.BlockSpec((bs, d), lambda i: (i, 0)),
            pl.BlockSpec((1, d), lambda i: (0, 0)),
            pl.BlockSpec((1, d), lambda i: (0, 0)),
        ],
        out_specs=[
            pl.BlockSpec((bs, dout), lambda i: (i, 0)),
            pl.BlockSpec((8, 128), lambda i: (0, 0)),
        ],
        out_shape=[
            jax.ShapeDtypeStruct((R, dout), jnp.float32),
            jax.ShapeDtypeStruct((8, 128), jnp.float32),
        ],
    )(X, g.reshape(1, d), b.reshape(1, d))


def _msg_call(G, Eh, scal, bs, with_g):
    R, d = Eh.shape
    grid = R // bs

    def body(*refs):
        if with_g:
            g_ref, e_ref, s_ref, o_ref = refs
            m = jnp.maximum(g_ref[...][:, :d] + e_ref[...], 0.0) + 1e-7
        else:
            e_ref, s_ref, o_ref = refs
            m = jnp.maximum(e_ref[...], 0.0) + 1e-7
        ex = jnp.exp(s_ref[0] * m - s_ref[1])
        o_ref[...] = jnp.concatenate([ex, m * ex], axis=1)

    in_specs = [pl.BlockSpec((bs, d), lambda i: (i, 0))]
    args = [Eh, scal]
    if with_g:
        gw = G.shape[1]
        in_specs = [pl.BlockSpec((bs, gw), lambda i: (i, 0))] + in_specs
        args = [G] + args
    in_specs.append(pl.BlockSpec(memory_space=pltpu.SMEM))
    return pl.pallas_call(
        body,
        grid=(grid,),
        in_specs=in_specs,
        out_specs=pl.BlockSpec((bs, 2 * d), lambda i: (i, 0)),
        out_shape=jax.ShapeDtypeStruct((R, 2 * d), jnp.float32),
    )(*args)


def _agg_div_call(P2, bs):
    _, NP, D = P2.shape
    d = D // 2
    grid = NP // bs

    def body(p_ref, o_ref):
        a = p_ref[0] + p_ref[1]
        o_ref[...] = jnp.concatenate(
            [a[:, d:] / (a[:, :d] + 1e-16), jnp.zeros((a.shape[0], d), jnp.float32)],
            axis=1)

    return pl.pallas_call(
        body,
        grid=(grid,),
        in_specs=[pl.BlockSpec((2, bs, D), lambda i: (0, i, 0))],
        out_specs=pl.BlockSpec((bs, D), lambda i: (i, 0)),
        out_shape=jax.ShapeDtypeStruct((NP, D), jnp.float32),
    )(P2)


def _fin_call(A, B, base, W1, b1, W2, b2, bs):
    R = B.shape[0]
    d = W1.shape[0]
    aw, bw = A.shape[1], B.shape[1]
    dh = W1.shape[1]
    dout = W2.shape[1]
    grid = R // bs

    def body(*refs):
        if base is not None:
            a_ref, b_ref, r_ref, w1_ref, b1_ref, w2_ref, b2_ref, o_ref = refs
        else:
            a_ref, b_ref, w1_ref, b1_ref, w2_ref, b2_ref, o_ref = refs
        h = a_ref[...][:, :d] + b_ref[...][:, :d]
        hh = jnp.maximum(
            jnp.dot(h, w1_ref[...], preferred_element_type=jnp.float32)
            + b1_ref[...], 0.0)
        o = jnp.dot(hh, w2_ref[...], preferred_element_type=jnp.float32) + b2_ref[...]
        if base is not None:
            o = r_ref[...] + o
        o_ref[...] = o

    in_specs = [
        pl.BlockSpec((bs, aw), lambda i: (i, 0)),
        pl.BlockSpec((bs, bw), lambda i: (i, 0)),
    ]
    args = [A, B]
    if base is not None:
        in_specs.append(pl.BlockSpec((bs, dout), lambda i: (i, 0)))
        args.append(base)
    in_specs += [
        pl.BlockSpec((d, dh), lambda i: (0, 0)),
        pl.BlockSpec((1, dh), lambda i: (0, 0)),
        pl.BlockSpec((dh, dout), lambda i: (0, 0)),
        pl.BlockSpec((1, dout), lambda i: (0, 0)),
    ]
    args += [W1, b1.reshape(1, dh), W2, b2.reshape(1, dout)]
    return pl.pallas_call(
        body,
        grid=(grid,),
        in_specs=in_specs,
        out_specs=pl.BlockSpec((bs, dout), lambda i: (i, 0)),
        out_shape=jax.ShapeDtypeStruct((R, dout), jnp.float32),
    )(*args)


_CH = 128


def _gather_rows(table, idxr):
    T, D = table.shape
    nch = idxr.shape[0]
    E = nch * _CH
    info = plsc.get_sparse_core_info()
    NC, NS = info.num_cores, info.num_subcores
    NW = NC * NS
    per_w = nch // NW
    mesh = plsc.VectorSubcoreMesh(core_axis_name="c", subcore_axis_name="s")

    @functools.partial(
        pl.kernel,
        mesh=mesh,
        out_type=jax.ShapeDtypeStruct((E, D), jnp.float32),
        scratch_types=[
            pltpu.VMEM((per_w, _CH), jnp.int32),
            pltpu.VMEM((2, _CH, D), jnp.float32),
            pltpu.SemaphoreType.DMA,
            pltpu.SemaphoreType.DMA,
            pltpu.SemaphoreType.DMA,
        ],
    )
    def k(table_hbm, idxr_hbm, out_hbm, idx_v, rows_v, sem_g, sem_w0, sem_w1):
        wid = lax.axis_index("s") * NC + lax.axis_index("c")
        pltpu.sync_copy(idxr_hbm.at[pl.ds(wid * per_w, per_w)], idx_v)
        sem_w = (sem_w0, sem_w1)
        wdesc = [None, None]
        for t in range(per_w):
            b = t % 2
            if wdesc[b] is not None:
                wdesc[b].wait()
            pltpu.async_copy(table_hbm.at[idx_v.at[t]], rows_v.at[b], sem_g).wait()
            wdesc[b] = pltpu.make_async_copy(
                rows_v.at[b],
                out_hbm.at[pl.ds((wid * per_w + t) * _CH, _CH)],
                sem_w[b])
            wdesc[b].start()
        wdesc[0].wait()
        if per_w > 1:
            wdesc[1].wait()

    return k(table, idxr)


def _scatter_add_rows(idxr, upd, zeros):
    E, D = upd.shape
    NP = zeros.shape[0]
    info = plsc.get_sparse_core_info()
    NC, NS = info.num_cores, info.num_subcores
    NW = NC * NS
    nch = E // _CH
    per_w = -(-nch // NW)
    stripe = NP // NS
    mesh = plsc.VectorSubcoreMesh(core_axis_name="c", subcore_axis_name="s")

    @functools.partial(
        pl.kernel,
        mesh=mesh,
        out_type=jax.ShapeDtypeStruct((NC, NP, D), jnp.float32),
        scratch_types=[
            pltpu.VMEM((2, _CH), jnp.int32),
            pltpu.VMEM((2, _CH, D), jnp.float32),
            pltpu.VMEM_SHARED((NP, D), jnp.float32),
            pltpu.SemaphoreType.DMA,
            pltpu.SemaphoreType.DMA,
            pltpu.SemaphoreType.DMA,
            pltpu.SemaphoreType.DMA,
        ],
    )
    def k(zeros_hbm, idxr_hbm, upd_hbm, out_hbm, idx_v, upd_v, acc_sh,
          sem_i0, sem_i1, sem_u0, sem_u1):
        cid = lax.axis_index("c")
        sid = lax.axis_index("s")
        wid = sid * NC + cid
        row0 = sid * stripe
        pltpu.sync_copy(zeros_hbm.at[pl.ds(row0, stripe)],
                        acc_sh.at[pl.ds(row0, stripe)])
        plsc.subcore_barrier()
        for t in range(per_w):
            b = t % 2
            c = wid + NW * t

            @pl.when(c < nch)
            def _():
                pltpu.sync_copy(idxr_hbm.at[c], idx_v.at[b])
                pltpu.sync_copy(upd_hbm.at[pl.ds(c * _CH, _CH)], upd_v.at[b])
                pltpu.sync_copy(upd_v.at[b], acc_sh.at[idx_v.at[b]], add=True)

        plsc.subcore_barrier()
        pltpu.sync_copy(acc_sh.at[pl.ds(row0, stripe)],
                        out_hbm.at[cid, pl.ds(row0, stripe)])

    return k(zeros, idxr, upd)


def _pad_cols(W, to):
    return jnp.zeros((W.shape[0], to), jnp.float32).at[:, :W.shape[1]].set(W)


def _pad_vec(b, to):
    return jnp.zeros((to,), jnp.float32).at[:b.shape[0]].set(b)


def _stab_const(beta, bound):
    return jnp.maximum(beta * (jnp.maximum(bound, 0.0) + 1e-7), 0.0)


def kernel(x, edge_index, edge_attr, params):
    N = x.shape[0]
    E = edge_index.shape[1]
    src = edge_index[0].astype(jnp.int32)
    dst = edge_index[1].astype(jnp.int32)

    k = src * N + dst
    ks_ = jnp.sort(k)
    keep = jnp.concatenate([jnp.ones((1,), bool), ks_[1:] != ks_[:-1]])
    rank = jnp.cumsum(keep) - 1
    idx = jnp.where(keep, rank.astype(jnp.int32), E)
    uk = jnp.zeros((E,), jnp.int32).at[idx].add(ks_, mode="drop",
                                                unique_indices=True)
    valid = jnp.arange(E) < keep.sum()
    row_c = uk // N
    col_c = uk % N
    seg = jnp.where(valid, col_c, N).astype(jnp.int32)
    rowg = jnp.where(valid, row_c, N + 1).astype(jnp.int32)

    pad = jnp.arange(3840, dtype=jnp.int32)
    src_r = jnp.concatenate([src, pad % N]).reshape(-1, _CH)
    rowg_r = jnp.concatenate([rowg, pad]).reshape(-1, _CH)
    dst_r = dst.reshape(-1, _CH)
    seg_r = seg.reshape(-1, _CH)

    zeros_acc = jnp.zeros((NPAD, ACC_D), jnp.float32)
    bs_n, bs_e = 2000, 2000

    pn, pe = params["node_enc"], params["edge_enc"]
    W2np = _pad_cols(pn["W2"], 128)
    b2np = _pad_vec(pn["b2"], 128)
    v, vmax8 = _row_mlp_call(x, pn["W1"], pn["b1"], W2np, b2np,
                             bs_n, want_max=True)
    e, emax8 = _row_mlp_call(edge_attr, pe["W1"], pe["b1"], pe["W2"], pe["b2"],
                             bs_e, want_max=True)

    for i, lp in enumerate(params["layers"]):
        if i == 0:
            vh, eh = v, e
            vmax, emax = jnp.max(vmax8), jnp.max(emax8)
        else:
            vh, vm8 = _ln_relu_call(v, lp["v_ln_g"], lp["v_ln_b"], bs_n,
                                    pad_to=128)
            eh, em8 = _ln_relu_call(e, lp["e_ln_g"], lp["e_ln_b"], bs_e)
            vmax, emax = jnp.max(vm8), jnp.max(em8)

        g = _gather_rows(vh, src_r)
        cv = _stab_const(lp["v_beta"], vmax + emax)
        Pv = _msg_call(g, eh, jnp.stack([lp["v_beta"], cv]), bs_e, with_g=True)
        partv = _scatter_add_rows(dst_r, Pv, zeros_acc)
        aggv = _agg_div_call(partv, 640)
        vmlp = lp["v_mlp"]
        v_new = _fin_call(aggv, vh, v if i > 0 else None,
                          vmlp["W1"], vmlp["b1"], vmlp["W2"], vmlp["b2"], bs_n)

        ce = _stab_const(lp["e_beta"], emax)
        Pe = _msg_call(None, eh, jnp.stack([lp["e_beta"], ce]), bs_e, with_g=False)
        parte = _scatter_add_rows(seg_r, Pe, zeros_acc)
        agge = _agg_div_call(parte, 640)
        ge = _gather_rows(agge, rowg_r)
        emlp = lp["e_mlp"]
        e_new = _fin_call(ge, eh, e if i > 0 else None,
                          emlp["W1"], emlp["b1"], emlp["W2"], emlp["b2"], bs_e)

        v, e = v_new, e_new

    pd_, pq = params["node_dec"], params["edge_dec"]
    dout = pd_["W2"].shape[1]
    node_out = _row_mlp_call(v, pd_["W1"], pd_["b1"],
                             _pad_cols(pd_["W2"], 128), _pad_vec(pd_["b2"], 128),
                             bs_n)[:, :dout]
    edge_out = _row_mlp_call(e, pq["W1"], pq["b1"],
                             _pad_cols(pq["W2"], 128), _pad_vec(pq["b2"], 128),
                             bs_e)[:, :dout]
    return node_out, edge_out
